# Initial kernel scaffold; baseline (speedup 1.0000x reference)
import jax
import jax.numpy as jnp
from jax.experimental import pallas as pl


def kernel(x, edge_index, W1a, b1a, W1b, b1b, W2a, b2a, W2b, b2b):
    raise NotImplementedError("write your pallas kernel here")


# speedup vs baseline: 11.9796x; 11.9796x over previous
import functools

import jax
import jax.numpy as jnp
from jax import lax
from jax.experimental import pallas as pl
from jax.experimental.pallas import tpu as pltpu
from jax.experimental.pallas import tpu_sc as plsc

_N = 10000
_E = 320000
_D = 128
_H = 16
_PACK = 8
_NPR = _N // _PACK

_NC, _NS = 2, 16
_NW = _NC * _NS
_CH = 128
_KCH = 80
_EP = _NW * _KCH * _CH
_NPAD = 10240
_TRASH = _N
_RD = _NPAD // _NS
_ZCH = _RD // _CH

_HIGH = lax.Precision.HIGHEST


def _sc_agg_body(p_hbm, src_hbm, dst_hbm, out_hbm,
                 src_v, dst_v, rows_v, sem, acc):
    c = lax.axis_index("c")
    s = lax.axis_index("s")
    wid = s * _NC + c

    def _zrow(i, carry):
        rows_v[i] = jnp.zeros((_H,), jnp.float32)
        return carry
    lax.fori_loop(0, _CH, _zrow, 0)

    def _zcopy(k, carry):
        pltpu.sync_copy(rows_v, acc.at[pl.ds(s * _RD + k * _CH, _CH)])
        return carry
    lax.fori_loop(0, _ZCH, _zcopy, 0)

    pltpu.sync_copy(src_hbm.at[pl.ds(wid * _KCH, _KCH)], src_v)
    pltpu.sync_copy(dst_hbm.at[pl.ds(wid * _KCH, _KCH)], dst_v)
    plsc.subcore_barrier()

    def _edge_chunk(j, carry):
        pltpu.async_copy(p_hbm.at[src_v.at[j]], rows_v, sem).wait()
        pltpu.sync_copy(rows_v, acc.at[dst_v.at[j]], add=True)
        return carry
    lax.fori_loop(0, _KCH, _edge_chunk, 0)
    plsc.subcore_barrier()

    pltpu.sync_copy(acc.at[pl.ds(s * _RD, _RD)],
                    out_hbm.at[c, pl.ds(s * _RD, _RD)])


@functools.lru_cache(maxsize=1)
def _sc_agg_call():
    return pl.kernel(
        _sc_agg_body,
        out_type=jax.ShapeDtypeStruct((_NC, _NPAD, _H), jnp.float32),
        mesh=plsc.VectorSubcoreMesh(core_axis_name="c", subcore_axis_name="s",
                                    num_cores=_NC, num_subcores=_NS),
        scratch_types=[
            pltpu.VMEM((_KCH, _CH), jnp.int32),
            pltpu.VMEM((_KCH, _CH), jnp.int32),
            pltpu.VMEM((_CH, _H), jnp.float32),
            pltpu.SemaphoreType.DMA,
            pltpu.VMEM_SHARED((_NPAD, _H), jnp.float32),
        ],
        compiler_params=pltpu.CompilerParams(use_tc_tiling_on_sc=False),
    )


def _sc_agg(p, src_p, dst_p):
    return _sc_agg_call()(p, src_p, dst_p)


def _tc1_body(x_ref, w_ref, o_ref):
    o_ref[...] = lax.dot_general(
        x_ref[...], w_ref[...], (((1,), (0,)), ((), ())),
        preferred_element_type=jnp.float32, precision=_HIGH)


def _tc2_body(p_ref, parts_ref, w1b_ref, b1a_ref, b1b_ref, w2a_ref, o_ref):
    a1 = parts_ref[0, : _NPR, :] + parts_ref[1, : _NPR, :]
    z1 = jnp.maximum(p_ref[...] + a1 + b1a_ref[...], 0.0)
    h = lax.dot_general(z1, w1b_ref[...], (((1,), (0,)), ((), ())),
                        preferred_element_type=jnp.float32, precision=_HIGH)
    h = jnp.maximum(h + b1b_ref[...], 0.0)
    o_ref[...] = lax.dot_general(h, w2a_ref[...], (((1,), (0,)), ((), ())),
                                 preferred_element_type=jnp.float32,
                                 precision=_HIGH)


def _tc3_body(q_ref, parts_ref, w2b_ref, b2a_ref, b2b_ref, o_ref):
    a2 = parts_ref[0, : _NPR, :] + parts_ref[1, : _NPR, :]
    z2 = jnp.maximum(q_ref[...] + a2 + b2a_ref[...], 0.0)
    op = lax.dot_general(z2, w2b_ref[...], (((1,), (0,)), ((), ())),
                         preferred_element_type=jnp.float32, precision=_HIGH)
    op = op + b2b_ref[...]
    segs = []
    for k in range(_PACK):
        seg = op[:, k * _D:(k + 1) * _D]
        m = jnp.max(seg, axis=1, keepdims=True)
        lse = jnp.log(jnp.sum(jnp.exp(seg - m), axis=1, keepdims=True)) + m
        segs.append(seg - lse)
    o_ref[...] = jnp.concatenate(segs, axis=1)


_tc1 = pl.pallas_call(
    _tc1_body,
    out_shape=jax.ShapeDtypeStruct((_NPR, _D), jnp.float32),
)

_tc2 = pl.pallas_call(
    _tc2_body,
    out_shape=jax.ShapeDtypeStruct((_NPR, _D), jnp.float32),
)

_tc3 = pl.pallas_call(
    _tc3_body,
    out_shape=jax.ShapeDtypeStruct((_NPR, _PACK * _D), jnp.float32),
)


def kernel(x, edge_index, W1a, b1a, W1b, b1b, W2a, b2a, W2b, b2b):
    pad = _EP - _E
    src_p = jnp.concatenate(
        [edge_index[0], jnp.zeros((pad,), jnp.int32)]).reshape(_EP // _CH, _CH)
    dst_p = jnp.concatenate(
        [edge_index[1], jnp.full((pad,), _TRASH, jnp.int32)]).reshape(
            _EP // _CH, _CH)

    eye = jnp.eye(_PACK, dtype=jnp.float32)
    w1a_bd = jnp.kron(eye, W1a)
    w1b_bd = jnp.kron(eye, W1b)
    w2a_bd = jnp.kron(eye, W2a)
    w2b_bd = jnp.kron(eye, W2b)
    b1a_t = jnp.tile(b1a, _PACK)[None]
    b1b_t = jnp.tile(b1b, _PACK)[None]
    b2a_t = jnp.tile(b2a, _PACK)[None]
    b2b_t = jnp.tile(b2b, _PACK)[None]

    xp = x.reshape(_NPR, _PACK * _D)

    p = _tc1(xp, w1a_bd)
    agg1 = _sc_agg(p.reshape(_N, _H), src_p, dst_p)
    agg1 = agg1.reshape(_NC, _NPAD // _PACK, _PACK * _H)
    q = _tc2(p, agg1, w1b_bd, b1a_t, b1b_t, w2a_bd)
    agg2 = _sc_agg(q.reshape(_N, _H), src_p, dst_p)
    agg2 = agg2.reshape(_NC, _NPAD // _PACK, _PACK * _H)
    outp = _tc3(q, agg2, w2b_bd, b2a_t, b2b_t)
    return outp.reshape(_N, _D)


# speedup vs baseline: 16.8208x; 1.4041x over previous
import functools

import jax
import jax.numpy as jnp
from jax import lax
from jax.experimental import pallas as pl
from jax.experimental.pallas import tpu as pltpu
from jax.experimental.pallas import tpu_sc as plsc

_N = 10000
_E = 320000
_D = 128
_H = 16
_PACK = 8
_NPR = _N // _PACK

_NC, _NS = 2, 16
_NW = _NC * _NS
_CH = 128
_KCH = 80
_EP = _NW * _KCH * _CH
_CHK = 8
_NCH = _KCH // _CHK
_NBUF = 3
_NPAD = 10240
_TRASH = _N
_RD = _NPAD // _NS
_ZCH = _RD // _CH

_HIGH = lax.Precision.HIGHEST


def _sc_agg_body(p_hbm, src_hbm, dst_hbm, out_hbm,
                 src_v, dst_v, rows_v, zrow_v, gsem, ssem, acc):
    c = lax.axis_index("c")
    s = lax.axis_index("s")
    wid = s * _NC + c

    def _zrow(i, carry):
        zrow_v[0, i] = jnp.zeros((_H,), jnp.float32)
        return carry
    lax.fori_loop(0, _CH, _zrow, 0)

    def _zcopy(k, carry):
        pltpu.sync_copy(
            zrow_v, acc.at[pl.ds(0, 1), pl.ds(s * _RD + k * _CH, _CH)])
        return carry
    lax.fori_loop(0, _ZCH, _zcopy, 0)

    pltpu.sync_copy(src_hbm.at[pl.ds(wid * _NCH, _NCH)], src_v)
    pltpu.sync_copy(dst_hbm.at[pl.ds(wid * _NCH, _NCH)], dst_v)
    plsc.subcore_barrier()

    def _gather(m, buf):
        pltpu.async_copy(
            p_hbm.at[src_v.at[pl.ds(m, 1)]], rows_v.at[buf], gsem)

    for m in range(_NBUF - 1):
        _gather(m, m)

    def _edge_chunk(j, carry):
        b = j % _NBUF
        pltpu.make_async_copy(
            p_hbm.at[src_v.at[pl.ds(0, 1)]], rows_v.at[b], gsem).wait()
        pltpu.async_copy(
            rows_v.at[b], acc.at[dst_v.at[pl.ds(j, 1)]], ssem, add=True)

        @pl.when(j >= 1)
        def _():
            pltpu.make_async_copy(
                rows_v.at[b], acc.at[dst_v.at[pl.ds(0, 1)]], ssem).wait()

        @pl.when(j + _NBUF - 1 < _NCH)
        def _():
            _gather(j + _NBUF - 1, (j + _NBUF - 1) % _NBUF)
        return carry
    lax.fori_loop(0, _NCH, _edge_chunk, 0)

    pltpu.make_async_copy(
        rows_v.at[0], acc.at[dst_v.at[pl.ds(0, 1)]], ssem).wait()
    plsc.subcore_barrier()

    pltpu.sync_copy(acc.at[pl.ds(0, 1), pl.ds(s * _RD, _RD)],
                    out_hbm.at[c, pl.ds(0, 1), pl.ds(s * _RD, _RD)])


@functools.lru_cache(maxsize=1)
def _sc_agg_call():
    return pl.kernel(
        _sc_agg_body,
        out_type=jax.ShapeDtypeStruct((_NC, 1, _NPAD, _H), jnp.float32),
        mesh=plsc.VectorSubcoreMesh(core_axis_name="c", subcore_axis_name="s",
                                    num_cores=_NC, num_subcores=_NS),
        scratch_types=[
            pltpu.VMEM((_NCH, _CHK * _CH), jnp.int32),
            pltpu.VMEM((_NCH, _CHK * _CH), jnp.int32),
            pltpu.VMEM((_NBUF, 1, _CHK * _CH, _H), jnp.float32),
            pltpu.VMEM((1, _CH, _H), jnp.float32),
            pltpu.SemaphoreType.DMA,
            pltpu.SemaphoreType.DMA,
            pltpu.VMEM_SHARED((1, _NPAD, _H), jnp.float32),
        ],
        compiler_params=pltpu.CompilerParams(use_tc_tiling_on_sc=False),
    )


def _sc_agg(p, src_p, dst_p):
    return _sc_agg_call()(p, src_p, dst_p)


def _tc1_body(x_ref, w_ref, o_ref):
    o_ref[...] = lax.dot_general(
        x_ref[...], w_ref[...], (((1,), (0,)), ((), ())),
        preferred_element_type=jnp.float32, precision=_HIGH)


def _tc2_body(p_ref, parts_ref, w1b_ref, b1a_ref, b1b_ref, w2a_ref, o_ref):
    a1 = parts_ref[0, : _NPR, :] + parts_ref[1, : _NPR, :]
    z1 = jnp.maximum(p_ref[...] + a1 + b1a_ref[...], 0.0)
    h = lax.dot_general(z1, w1b_ref[...], (((1,), (0,)), ((), ())),
                        preferred_element_type=jnp.float32, precision=_HIGH)
    h = jnp.maximum(h + b1b_ref[...], 0.0)
    o_ref[...] = lax.dot_general(h, w2a_ref[...], (((1,), (0,)), ((), ())),
                                 preferred_element_type=jnp.float32,
                                 precision=_HIGH)


def _tc3_body(q_ref, parts_ref, w2b_ref, b2a_ref, b2b_ref, o_ref):
    a2 = parts_ref[0, : _NPR, :] + parts_ref[1, : _NPR, :]
    z2 = jnp.maximum(q_ref[...] + a2 + b2a_ref[...], 0.0)
    op = lax.dot_general(z2, w2b_ref[...], (((1,), (0,)), ((), ())),
                         preferred_element_type=jnp.float32, precision=_HIGH)
    op = op + b2b_ref[...]
    segs = []
    for k in range(_PACK):
        seg = op[:, k * _D:(k + 1) * _D]
        m = jnp.max(seg, axis=1, keepdims=True)
        lse = jnp.log(jnp.sum(jnp.exp(seg - m), axis=1, keepdims=True)) + m
        segs.append(seg - lse)
    o_ref[...] = jnp.concatenate(segs, axis=1)


_tc1 = pl.pallas_call(
    _tc1_body,
    out_shape=jax.ShapeDtypeStruct((_NPR, _D), jnp.float32),
)

_tc2 = pl.pallas_call(
    _tc2_body,
    out_shape=jax.ShapeDtypeStruct((_NPR, _D), jnp.float32),
)

_tc3 = pl.pallas_call(
    _tc3_body,
    out_shape=jax.ShapeDtypeStruct((_NPR, _PACK * _D), jnp.float32),
)


def kernel(x, edge_index, W1a, b1a, W1b, b1b, W2a, b2a, W2b, b2b):
    pad = _EP - _E
    chunk = _CHK * _CH
    src_p = jnp.concatenate(
        [edge_index[0], jnp.zeros((pad,), jnp.int32)]).reshape(
            _EP // chunk, chunk)
    dst_p = jnp.concatenate(
        [edge_index[1], jnp.full((pad,), _TRASH, jnp.int32)]).reshape(
            _EP // chunk, chunk)

    eye = jnp.eye(_PACK, dtype=jnp.float32)
    w1a_bd = jnp.kron(eye, W1a)
    w1b_bd = jnp.kron(eye, W1b)
    w2a_bd = jnp.kron(eye, W2a)
    w2b_bd = jnp.kron(eye, W2b)
    b1a_t = jnp.tile(b1a, _PACK)[None]
    b1b_t = jnp.tile(b1b, _PACK)[None]
    b2a_t = jnp.tile(b2a, _PACK)[None]
    b2b_t = jnp.tile(b2b, _PACK)[None]

    xp = x.reshape(_NPR, _PACK * _D)

    p = _tc1(xp, w1a_bd)
    agg1 = _sc_agg(p.reshape(1, _N, _H), src_p, dst_p)
    agg1 = agg1.reshape(_NC, _NPAD // _PACK, _PACK * _H)
    q = _tc2(p, agg1, w1b_bd, b1a_t, b1b_t, w2a_bd)
    agg2 = _sc_agg(q.reshape(1, _N, _H), src_p, dst_p)
    agg2 = agg2.reshape(_NC, _NPAD // _PACK, _PACK * _H)
    outp = _tc3(q, agg2, w2b_bd, b2a_t, b2b_t)
    return outp.reshape(_N, _D)
